# baseline (speedup 1.0000x reference)
import functools

import jax
import jax.numpy as jnp
from jax import lax
from jax.experimental import pallas as pl
from jax.experimental.pallas import tpu as pltpu
from jax.experimental.pallas import tpu_sc as plsc

H = 128
EPS = 1e-5
NC, NS, L = 2, 16, 16
NW = NC * NS
CH = 128
NG = 50000
QR = 12504
CS = 80
SROWS = 12560


def _bdot(a, b):
    return jnp.dot(a.astype(jnp.bfloat16), b.astype(jnp.bfloat16),
                   preferred_element_type=jnp.float32)


def _mm_body(x_ref, w_ref, o_ref):
    o_ref[...] = _bdot(x_ref[...], w_ref[...])


def _matmul(x, w, bn):
    n, k = x.shape
    m = w.shape[1]
    return pl.pallas_call(
        _mm_body,
        grid=(n // bn,),
        in_specs=[pl.BlockSpec((bn, k), lambda i: (i, 0)),
                  pl.BlockSpec((k, m), lambda i: (0, 0))],
        out_specs=pl.BlockSpec((bn, m), lambda i: (i, 0)),
        out_shape=jax.ShapeDtypeStruct((n, m), jnp.float32),
    )(x, w)


def _ln_tc(h, g, b):
    mu = jnp.mean(h, axis=-1, keepdims=True)
    var = jnp.mean((h - mu) * (h - mu), axis=-1, keepdims=True)
    return (h - mu) * lax.rsqrt(var + EPS) * g + b


def _silu(x):
    return x * jax.nn.sigmoid(x)


def _edge_body(em_ref, g_ref, w1_ref, b1_ref, lg_ref, lb_ref, w2_ref, b2_ref,
               o_ref):
    em = em_ref[...]
    pre = _bdot(em, w1_ref[...]) + g_ref[...] + b1_ref[...]
    h = _ln_tc(_silu(pre), lg_ref[...], lb_ref[...])
    o_ref[...] = em + _bdot(h, w2_ref[...]) + b2_ref[...]


def _edge_mlp(em, g, w1, b1, lg, lb, w2, b2, be):
    n = em.shape[0]
    vec = lambda v: v.reshape(1, H)
    row_spec = pl.BlockSpec((1, H), lambda i: (0, 0))
    return pl.pallas_call(
        _edge_body,
        grid=(n // be,),
        in_specs=[pl.BlockSpec((be, H), lambda i: (i, 0)),
                  pl.BlockSpec((be, H), lambda i: (i, 0)),
                  pl.BlockSpec((H, H), lambda i: (0, 0)),
                  row_spec, row_spec, row_spec,
                  pl.BlockSpec((H, H), lambda i: (0, 0)),
                  row_spec],
        out_specs=pl.BlockSpec((be, H), lambda i: (i, 0)),
        out_shape=jax.ShapeDtypeStruct((n, H), jnp.float32),
    )(em, g, w1, vec(b1), vec(lg), vec(lb), w2, vec(b2))


def _node_body(vg_ref, agg_ref, w1a_ref, w1b_ref, b1_ref, lg_ref, lb_ref,
               w2_ref, b2_ref, ow1_ref, ob1_ref, ow2_ref, ob2_ref, o_ref):
    vg = vg_ref[...]
    pre = (_bdot(vg, w1a_ref[...]) + _bdot(agg_ref[...], w1b_ref[...])
           + b1_ref[...])
    h = _ln_tc(_silu(pre), lg_ref[...], lb_ref[...])
    vg2 = vg + _bdot(h, w2_ref[...]) + b2_ref[...]
    h2 = _silu(_bdot(vg2, ow1_ref[...]) + ob1_ref[...])
    o_ref[...] = _bdot(h2, ow2_ref[...]) + ob2_ref[...]


def _node_mlp(vg, agg, w1a, w1b, b1, lg, lb, w2, b2, ow1, ob1, ow2, ob2, bn):
    n = vg.shape[0]
    out = ow2.shape[1]
    vec = lambda v: v.reshape(1, -1)
    row_spec = pl.BlockSpec((1, H), lambda i: (0, 0))
    orow_spec = pl.BlockSpec((1, out), lambda i: (0, 0))
    return pl.pallas_call(
        _node_body,
        grid=(n // bn,),
        in_specs=[pl.BlockSpec((bn, H), lambda i: (i, 0)),
                  pl.BlockSpec((bn, H), lambda i: (i, 0)),
                  pl.BlockSpec((H, H), lambda i: (0, 0)),
                  pl.BlockSpec((H, H), lambda i: (0, 0)),
                  row_spec, row_spec, row_spec,
                  pl.BlockSpec((H, H), lambda i: (0, 0)),
                  row_spec,
                  pl.BlockSpec((H, out), lambda i: (0, 0)),
                  orow_spec,
                  pl.BlockSpec((H, out), lambda i: (0, 0)),
                  orow_spec],
        out_specs=pl.BlockSpec((bn, out), lambda i: (i, 0)),
        out_shape=jax.ShapeDtypeStruct((n, out), jnp.float32),
    )(vg, agg, w1a, w1b, vec(b1), vec(lg), vec(lb), w2, vec(b2),
      ow1, vec(ob1), ow2, vec(ob2))


def _sc_gather_sum(tS, tR, send_p, recv_p):
    ep = send_p.shape[0]
    nch = ep // CH
    per_w = nch // NW
    npair = per_w // 2
    mesh = plsc.VectorSubcoreMesh(core_axis_name="c", subcore_axis_name="s")

    @functools.partial(
        pl.kernel,
        out_type=jax.ShapeDtypeStruct((ep, H), jnp.float32),
        mesh=mesh,
        scratch_types=[pltpu.VMEM((per_w * CH,), jnp.int32),
                       pltpu.VMEM((per_w * CH,), jnp.int32),
                       pltpu.VMEM((CH, H), jnp.float32),
                       pltpu.VMEM((CH, H), jnp.float32),
                       pltpu.VMEM((CH, H), jnp.float32),
                       pltpu.VMEM((CH, H), jnp.float32),
                       pltpu.SemaphoreType.DMA,
                       pltpu.SemaphoreType.DMA,
                       pltpu.SemaphoreType.DMA,
                       pltpu.SemaphoreType.DMA],
    )
    def k(ts_hbm, tr_hbm, s_hbm, r_hbm, g_hbm, idxa, idxb,
          a0, b0, a1, b1, sa0, sb0, sa1, sb1):
        wid = lax.axis_index("s") * NC + lax.axis_index("c")
        wbase = wid * per_w * CH
        pltpu.sync_copy(s_hbm.at[pl.ds(wbase, per_w * CH)], idxa)
        pltpu.sync_copy(r_hbm.at[pl.ds(wbase, per_w * CH)], idxb)

        def start(i, abuf, bbuf, asem, bsem):
            off = pl.ds(i * CH, CH)
            pltpu.async_copy(ts_hbm.at[idxa.at[off]], abuf, asem)
            pltpu.async_copy(tr_hbm.at[idxb.at[off]], bbuf, bsem)

        def finish(i, abuf, bbuf, asem, bsem):
            pltpu.make_async_copy(ts_hbm.at[idxa.at[pl.ds(0, CH)]],
                                  abuf, asem).wait()
            pltpu.make_async_copy(tr_hbm.at[idxb.at[pl.ds(0, CH)]],
                                  bbuf, bsem).wait()

            @pl.loop(0, CH)
            def _(rr):
                @pl.loop(0, H, step=L)
                def _(jj):
                    slc = (pl.ds(rr, 1), pl.ds(jj, L))
                    abuf.at[slc][...] = abuf.at[slc][...] + bbuf.at[slc][...]

            pltpu.sync_copy(abuf, g_hbm.at[pl.ds(wbase + i * CH, CH)])

        start(0, a0, b0, sa0, sb0)

        @pl.loop(0, npair)
        def _(j):
            c0 = 2 * j
            start(c0 + 1, a1, b1, sa1, sb1)
            finish(c0, a0, b0, sa0, sb0)

            @pl.when(j + 1 < npair)
            def _():
                start(c0 + 2, a0, b0, sa0, sb0)

            finish(c0 + 1, a1, b1, sa1, sb1)

    return k(tS, tR, send_p, recv_p)


def _sc_scatter_add(e, recv_p):
    ep = e.shape[0]
    mesh = plsc.VectorSubcoreMesh(core_axis_name="c", subcore_axis_name="s")

    per_s = ep // NS // CS
    npair = per_s // 2

    @functools.partial(
        pl.kernel,
        out_type=jax.ShapeDtypeStruct((NG, H), jnp.float32),
        mesh=mesh,
        scratch_types=[pltpu.VMEM((CS,), jnp.int32),
                       pltpu.VMEM((CS,), jnp.int32),
                       pltpu.VMEM((CS, H), jnp.float32),
                       pltpu.VMEM((CS, H), jnp.float32),
                       pltpu.VMEM_SHARED((SROWS, H), jnp.float32),
                       pltpu.SemaphoreType.DMA,
                       pltpu.SemaphoreType.DMA,
                       pltpu.SemaphoreType.DMA,
                       pltpu.SemaphoreType.DMA],
    )
    def k(e_hbm, r_hbm, agg_hbm, idx0, idx1, e0, e1, acc,
          se0, se1, si0, si1):
        cid = lax.axis_index("c")
        sid = lax.axis_index("s")
        sbase = sid * per_s * CS

        def start(i, ebuf, ibuf, se, si):
            pltpu.async_copy(r_hbm.at[pl.ds(sbase + i * CS, CS)], ibuf, si)
            pltpu.async_copy(e_hbm.at[pl.ds(sbase + i * CS, CS)], ebuf, se)

        @pl.loop(0, 2)
        def _(p):
            q = 2 * p + cid
            lo = q * QR
            cnt = jnp.where(q < 3, QR, NG - 3 * QR)
            @pl.loop(0, CS)
            def _(rr):
                @pl.loop(0, H, step=L)
                def _(jj):
                    e0.at[pl.ds(rr, 1), pl.ds(jj, L)][...] = (
                        jnp.zeros((1, L), jnp.float32))

            @pl.loop(sid, SROWS // CS, step=NS)
            def _(b):
                pltpu.sync_copy(e0, acc.at[pl.ds(b * CS, CS)])

            plsc.subcore_barrier()

            def finish(i, ebuf, ibuf, se, si):
                pltpu.make_async_copy(r_hbm.at[pl.ds(0, CS)], ibuf,
                                      si).wait()

                @pl.loop(0, CS, step=L)
                def _(jj):
                    v = ibuf.at[pl.ds(jj, L)][...]
                    lv = v - lo
                    ok = (lv >= 0) & (lv < cnt)
                    ibuf.at[pl.ds(jj, L)][...] = jnp.where(ok, lv, QR + sid)

                pltpu.make_async_copy(e_hbm.at[pl.ds(0, CS)], ebuf,
                                      se).wait()
                pltpu.sync_copy(ebuf, acc.at[ibuf], add=True)

            start(0, e0, idx0, se0, si0)

            @pl.loop(0, npair)
            def _(j):
                c0 = 2 * j
                start(c0 + 1, e1, idx1, se1, si1)
                finish(c0, e0, idx0, se0, si0)

                @pl.when(j + 1 < npair)
                def _():
                    start(c0 + 2, e0, idx0, se0, si0)

                finish(c0 + 1, e1, idx1, se1, si1)

            plsc.subcore_barrier()

            @pl.loop(sid, cnt // 8, step=NS)
            def _(b):
                pltpu.sync_copy(acc.at[pl.ds(b * 8, 8)],
                                agg_hbm.at[pl.ds(lo + b * 8, 8)])

            plsc.subcore_barrier()

    return k(e, recv_p)


def kernel(vM, vG, eM2G, senders, receivers, e_W1, e_b1, e_g, e_beta, e_W2,
           e_b2, n_W1, n_b1, n_g, n_beta, n_W2, n_b2, o_W1, o_b1, o_W2, o_b2):
    vm0 = vM[0]
    vg0 = vG[0]
    E = senders.shape[0]
    ep = ((E + NW * CH - 1) // (NW * CH)) * (NW * CH)
    pad = ep - E

    w1e, w1m, w1g = e_W1[:H], e_W1[H:2 * H], e_W1[2 * H:]
    nw1a, nw1b = n_W1[:H], n_W1[H:]

    send_p = jnp.concatenate([senders, jnp.zeros((pad,), jnp.int32)])
    recv_g = jnp.concatenate([receivers, jnp.zeros((pad,), jnp.int32)])
    recv_s = jnp.concatenate(
        [receivers, jnp.full((pad,), jnp.int32(1 << 30))])
    em_p = jnp.concatenate([eM2G, jnp.zeros((pad, H), jnp.float32)])

    tS = _matmul(vm0, w1m, 2000)
    tR = _matmul(vg0, w1g, 2000)

    G = _sc_gather_sum(tS, tR, send_p, recv_g)
    e = _edge_mlp(em_p, G, w1e, e_b1, e_g, e_beta, e_W2, e_b2, 2048)
    agg = _sc_scatter_add(e, recv_s)
    out = _node_mlp(vg0, agg, nw1a, nw1b, n_b1, n_g, n_beta, n_W2, n_b2,
                    o_W1, o_b1, o_W2, o_b2, 2000)
    return out[None]


# speedup vs baseline: 1.1268x; 1.1268x over previous
import functools

import jax
import jax.numpy as jnp
from jax import lax
from jax.experimental import pallas as pl
from jax.experimental.pallas import tpu as pltpu
from jax.experimental.pallas import tpu_sc as plsc

H = 128
EPS = 1e-5
NC, NS, L = 2, 16, 16
NW = NC * NS
CH = 128
NG = 50000
QR = 12504
Q3 = NG - 3 * QR
BC = 96
CAPW = 5216
SROWS = 12576


def _bdot(a, b):
    return jnp.dot(a, b, preferred_element_type=jnp.float32)


def _mm_body(x_ref, w_ref, o_ref):
    o_ref[...] = _bdot(x_ref[...], w_ref[...])


def _matmul(x, w, bn):
    n, k = x.shape
    m = w.shape[1]
    return pl.pallas_call(
        _mm_body,
        grid=(n // bn,),
        in_specs=[pl.BlockSpec((bn, k), lambda i: (i, 0)),
                  pl.BlockSpec((k, m), lambda i: (0, 0))],
        out_specs=pl.BlockSpec((bn, m), lambda i: (i, 0)),
        out_shape=jax.ShapeDtypeStruct((n, m), jnp.float32),
    )(x, w)


def _ln_tc(h, g, b):
    mu = jnp.mean(h, axis=-1, keepdims=True)
    var = jnp.mean((h - mu) * (h - mu), axis=-1, keepdims=True)
    return (h - mu) * lax.rsqrt(var + EPS) * g + b


def _silu(x):
    return x * jax.nn.sigmoid(x)


def _edge_body(em_ref, g_ref, w1_ref, b1_ref, lg_ref, lb_ref, w2_ref, b2_ref,
               o_ref):
    em = em_ref[...]
    pre = _bdot(em, w1_ref[...]) + g_ref[...] + b1_ref[...]
    h = _ln_tc(_silu(pre), lg_ref[...], lb_ref[...])
    o_ref[...] = em + _bdot(h, w2_ref[...]) + b2_ref[...]


def _edge_mlp(em, g, w1, b1, lg, lb, w2, b2, be):
    n = em.shape[0]
    vec = lambda v: v.reshape(1, H)
    row_spec = pl.BlockSpec((1, H), lambda i: (0, 0))
    return pl.pallas_call(
        _edge_body,
        grid=(n // be,),
        in_specs=[pl.BlockSpec((be, H), lambda i: (i, 0)),
                  pl.BlockSpec((be, H), lambda i: (i, 0)),
                  pl.BlockSpec((H, H), lambda i: (0, 0)),
                  row_spec, row_spec, row_spec,
                  pl.BlockSpec((H, H), lambda i: (0, 0)),
                  row_spec],
        out_specs=pl.BlockSpec((be, H), lambda i: (i, 0)),
        out_shape=jax.ShapeDtypeStruct((n, H), jnp.float32),
    )(em, g, w1, vec(b1), vec(lg), vec(lb), w2, vec(b2))


def _node_body(vg_ref, agg_ref, w1a_ref, w1b_ref, b1_ref, lg_ref, lb_ref,
               w2_ref, b2_ref, ow1_ref, ob1_ref, ow2_ref, ob2_ref, o_ref):
    vg = vg_ref[...]
    pre = (_bdot(vg, w1a_ref[...]) + _bdot(agg_ref[...], w1b_ref[...])
           + b1_ref[...])
    h = _ln_tc(_silu(pre), lg_ref[...], lb_ref[...])
    vg2 = vg + _bdot(h, w2_ref[...]) + b2_ref[...]
    h2 = _silu(_bdot(vg2, ow1_ref[...]) + ob1_ref[...])
    o_ref[...] = _bdot(h2, ow2_ref[...]) + ob2_ref[...]


def _node_mlp(vg, agg, w1a, w1b, b1, lg, lb, w2, b2, ow1, ob1, ow2, ob2, bn):
    n = vg.shape[0]
    out = ow2.shape[1]
    vec = lambda v: v.reshape(1, -1)
    row_spec = pl.BlockSpec((1, H), lambda i: (0, 0))
    orow_spec = pl.BlockSpec((1, out), lambda i: (0, 0))
    return pl.pallas_call(
        _node_body,
        grid=(n // bn,),
        in_specs=[pl.BlockSpec((bn, H), lambda i: (i, 0)),
                  pl.BlockSpec((bn, H), lambda i: (i, 0)),
                  pl.BlockSpec((H, H), lambda i: (0, 0)),
                  pl.BlockSpec((H, H), lambda i: (0, 0)),
                  row_spec, row_spec, row_spec,
                  pl.BlockSpec((H, H), lambda i: (0, 0)),
                  row_spec,
                  pl.BlockSpec((H, out), lambda i: (0, 0)),
                  orow_spec,
                  pl.BlockSpec((H, out), lambda i: (0, 0)),
                  orow_spec],
        out_specs=pl.BlockSpec((bn, out), lambda i: (i, 0)),
        out_shape=jax.ShapeDtypeStruct((n, out), jnp.float32),
    )(vg, agg, w1a, w1b, vec(b1), vec(lg), vec(lb), w2, vec(b2),
      ow1, vec(ob1), ow2, vec(ob2))


def _sc_gather_sum(tS, tR, send_p, recv_p):
    ep = send_p.shape[0]
    nch = ep // CH
    per_w = nch // NW
    npair = per_w // 2
    mesh = plsc.VectorSubcoreMesh(core_axis_name="c", subcore_axis_name="s")

    @functools.partial(
        pl.kernel,
        out_type=jax.ShapeDtypeStruct((ep, H), jnp.float32),
        mesh=mesh,
        scratch_types=[pltpu.VMEM((per_w * CH,), jnp.int32),
                       pltpu.VMEM((per_w * CH,), jnp.int32),
                       pltpu.VMEM((CH, H), jnp.float32),
                       pltpu.VMEM((CH, H), jnp.float32),
                       pltpu.VMEM((CH, H), jnp.float32),
                       pltpu.VMEM((CH, H), jnp.float32),
                       pltpu.SemaphoreType.DMA,
                       pltpu.SemaphoreType.DMA,
                       pltpu.SemaphoreType.DMA,
                       pltpu.SemaphoreType.DMA],
    )
    def k(ts_hbm, tr_hbm, s_hbm, r_hbm, g_hbm, idxa, idxb,
          a0, b0, a1, b1, sa0, sb0, sa1, sb1):
        wid = lax.axis_index("s") * NC + lax.axis_index("c")
        wbase = wid * per_w * CH
        pltpu.sync_copy(s_hbm.at[pl.ds(wbase, per_w * CH)], idxa)
        pltpu.sync_copy(r_hbm.at[pl.ds(wbase, per_w * CH)], idxb)

        def start(i, abuf, bbuf, asem, bsem):
            off = pl.ds(i * CH, CH)
            pltpu.async_copy(ts_hbm.at[idxa.at[off]], abuf, asem)
            pltpu.async_copy(tr_hbm.at[idxb.at[off]], bbuf, bsem)

        def finish(i, abuf, bbuf, asem, bsem):
            pltpu.make_async_copy(ts_hbm.at[idxa.at[pl.ds(0, CH)]],
                                  abuf, asem).wait()
            pltpu.make_async_copy(tr_hbm.at[idxb.at[pl.ds(0, CH)]],
                                  bbuf, bsem).wait()

            @pl.loop(0, CH)
            def _(rr):
                @pl.loop(0, H, step=L)
                def _(jj):
                    slc = (pl.ds(rr, 1), pl.ds(jj, L))
                    abuf.at[slc][...] = abuf.at[slc][...] + bbuf.at[slc][...]

            pltpu.sync_copy(abuf, g_hbm.at[pl.ds(wbase + i * CH, CH)])

        start(0, a0, b0, sa0, sb0)

        @pl.loop(0, npair)
        def _(j):
            c0 = 2 * j
            start(c0 + 1, a1, b1, sa1, sb1)
            finish(c0, a0, b0, sa0, sb0)

            @pl.when(j + 1 < npair)
            def _():
                start(c0 + 2, a0, b0, sa0, sb0)

            finish(c0 + 1, a1, b1, sa1, sb1)

    return k(tS, tR, send_p, recv_p)


def _sc_bucket(recv_b):
    ep = recv_b.shape[0]
    per_w = ep // NW
    mesh = plsc.VectorSubcoreMesh(core_axis_name="c", subcore_axis_name="s")

    @functools.partial(
        pl.kernel,
        out_type=(jax.ShapeDtypeStruct((NW * 4 * CAPW,), jnp.int32),
                  jax.ShapeDtypeStruct((NW * 4 * CAPW,), jnp.int32),
                  jax.ShapeDtypeStruct((NW * L,), jnp.int32)),
        mesh=mesh,
        scratch_types=[pltpu.VMEM((per_w,), jnp.int32),
                       pltpu.VMEM((4 * CAPW,), jnp.int32),
                       pltpu.VMEM((4 * CAPW,), jnp.int32),
                       pltpu.VMEM((L,), jnp.int32),
                       pltpu.SMEM((8,), jnp.int32)],
        compiler_params=pltpu.CompilerParams(needs_layout_passes=False),
    )
    def k(r_hbm, ids_hbm, lxs_hbm, cnts_hbm, rbuf, bid, blx, cbuf, offs):
        wid = lax.axis_index("s") * NC + lax.axis_index("c")
        wbase = wid * per_w
        pltpu.sync_copy(r_hbm.at[pl.ds(wbase, per_w)], rbuf)
        lane = lax.iota(jnp.int32, L)
        for q in range(4):
            offs[q] = 0

        @pl.loop(0, per_w, step=L)
        def _(i):
            rv = rbuf.at[pl.ds(i, L)][...]
            idv = wbase + i + lane
            for q in range(4):
                lv = rv - q * QR
                cq = QR if q < 3 else Q3
                m = (lv >= 0) & (lv < cq)
                off = offs[q]
                mi = m.astype(jnp.int32)
                pos = q * CAPW + off + lax.cumsum(mi) - 1
                plsc.store_scatter(bid, [pos], idv, mask=m)
                plsc.store_scatter(blx, [pos], lv, mask=m)
                offs[q] = off + jnp.sum(mi)

        cvec = jnp.zeros((L,), jnp.int32)
        for q in range(4):
            off = offs[q]
            for t in range(BC // L):
                bid.at[pl.ds(q * CAPW + off + t * L, L)][...] = (
                    jnp.full((L,), 0, jnp.int32) + wbase)
                blx.at[pl.ds(q * CAPW + off + t * L, L)][...] = (
                    jnp.full((L,), QR, jnp.int32))
            nchq = (jnp.full((L,), BC - 1, jnp.int32) + off) // BC
            cvec = cvec + jnp.where(lane == q, nchq, 0)
            pltpu.sync_copy(bid.at[pl.ds(q * CAPW, CAPW)],
                            ids_hbm.at[pl.ds((wid * 4 + q) * CAPW, CAPW)])
            pltpu.sync_copy(blx.at[pl.ds(q * CAPW, CAPW)],
                            lxs_hbm.at[pl.ds((wid * 4 + q) * CAPW, CAPW)])
        cbuf[...] = cvec
        pltpu.sync_copy(cbuf, cnts_hbm.at[pl.ds(wid * L, L)])

    return k(recv_b)


def _sc_scatter_add(e, ids, lxs, cnts):
    mesh = plsc.VectorSubcoreMesh(core_axis_name="c", subcore_axis_name="s")

    @functools.partial(
        pl.kernel,
        out_type=jax.ShapeDtypeStruct((NG, H), jnp.float32),
        mesh=mesh,
        scratch_types=[pltpu.VMEM((BC,), jnp.int32),
                       pltpu.VMEM((BC,), jnp.int32),
                       pltpu.VMEM((BC,), jnp.int32),
                       pltpu.VMEM((BC,), jnp.int32),
                       pltpu.VMEM((L,), jnp.int32),
                       pltpu.VMEM((L,), jnp.int32),
                       pltpu.VMEM((BC, H), jnp.float32),
                       pltpu.VMEM((BC, H), jnp.float32),
                       pltpu.VMEM_SHARED((SROWS, H), jnp.float32),
                       pltpu.SemaphoreType.DMA,
                       pltpu.SemaphoreType.DMA],
        compiler_params=pltpu.CompilerParams(needs_layout_passes=False),
    )
    def k(e_hbm, ids_hbm, lxs_hbm, cnts_hbm, z_hbm, agg_hbm,
          id0, id1, lx0, lx1, cb0, cb1, e0, e1, acc, se0, se1):
        cid = lax.axis_index("c")
        sid = lax.axis_index("s")
        lane = lax.iota(jnp.int32, L)

        @pl.loop(0, 2)
        def _(p):
            q = 2 * p + cid
            lo = q * QR
            cnt = jnp.where(q < 3, QR, Q3)

            @pl.loop(sid, SROWS // BC, step=NS)
            def _(b):
                pltpu.sync_copy(z_hbm, acc.at[pl.ds(b * BC, BC)])

            plsc.subcore_barrier()

            w0 = 2 * sid
            w1 = w0 + 1
            pltpu.sync_copy(cnts_hbm.at[pl.ds(w0 * L, L)], cb0)
            pltpu.sync_copy(cnts_hbm.at[pl.ds(w1 * L, L)], cb1)
            n0 = jnp.max(jnp.where(lane == q, cb0[...], 0))
            n1 = jnp.max(jnp.where(lane == q, cb1[...], 0))
            total = n0 + n1

            def seg_off(i):
                in0 = i < n0
                w = jnp.where(in0, w0, w1)
                c = jnp.where(in0, i, i - n0)
                return (w * 4 + q) * CAPW + c * BC

            def start(i, ebuf, idb, lxb, se):
                off = seg_off(i)
                pltpu.sync_copy(ids_hbm.at[pl.ds(off, BC)], idb)
                pltpu.sync_copy(lxs_hbm.at[pl.ds(off, BC)], lxb)
                pltpu.async_copy(e_hbm.at[idb], ebuf, se)

            def finish(i, ebuf, idb, lxb, se):
                pltpu.make_async_copy(e_hbm.at[pl.ds(0, BC)], ebuf,
                                      se).wait()
                pltpu.sync_copy(ebuf, acc.at[lxb], add=True)

            npr = lax.shift_right_logical(total, 1)
            odd = total - 2 * npr

            @pl.when(total > 0)
            def _():
                start(0, e0, id0, lx0, se0)

            @pl.loop(0, npr)
            def _(j):
                c0 = 2 * j
                start(c0 + 1, e1, id1, lx1, se1)
                finish(c0, e0, id0, lx0, se0)

                @pl.when(c0 + 2 < total)
                def _():
                    start(c0 + 2, e0, id0, lx0, se0)

                finish(c0 + 1, e1, id1, lx1, se1)

            @pl.when(odd == 1)
            def _():
                finish(total - 1, e0, id0, lx0, se0)

            plsc.subcore_barrier()

            @pl.loop(sid, cnt // 8, step=NS)
            def _(b):
                pltpu.sync_copy(acc.at[pl.ds(b * 8, 8)],
                                agg_hbm.at[pl.ds(lo + b * 8, 8)])

            plsc.subcore_barrier()

    return k(e, ids, lxs, cnts, jnp.zeros((BC, H), jnp.float32))


def kernel(vM, vG, eM2G, senders, receivers, e_W1, e_b1, e_g, e_beta, e_W2,
           e_b2, n_W1, n_b1, n_g, n_beta, n_W2, n_b2, o_W1, o_b1, o_W2, o_b2):
    vm0 = vM[0]
    vg0 = vG[0]
    E = senders.shape[0]
    ep = ((E + NW * CH - 1) // (NW * CH)) * (NW * CH)
    pad = ep - E

    w1e, w1m, w1g = e_W1[:H], e_W1[H:2 * H], e_W1[2 * H:]
    nw1a, nw1b = n_W1[:H], n_W1[H:]

    send_p = jnp.concatenate([senders, jnp.zeros((pad,), jnp.int32)])
    recv_g = jnp.concatenate([receivers, jnp.zeros((pad,), jnp.int32)])
    recv_s = jnp.concatenate(
        [receivers, jnp.full((pad,), jnp.int32(1 << 30))])
    em_p = jnp.concatenate([eM2G, jnp.zeros((pad, H), jnp.float32)])

    tS = _matmul(vm0, w1m, 2000)
    tR = _matmul(vg0, w1g, 2000)

    ids, lxs, cnts = _sc_bucket(recv_s)
    G = _sc_gather_sum(tS, tR, send_p, recv_g)
    e = _edge_mlp(em_p, G, w1e, e_b1, e_g, e_beta, e_W2, e_b2, 2048)
    agg = _sc_scatter_add(e, ids, lxs, cnts)
    out = _node_mlp(vg0, agg, nw1a, nw1b, n_b1, n_g, n_beta, n_W2, n_b2,
                    o_W1, o_b1, o_W2, o_b2, 2000)
    return out[None]


# speedup vs baseline: 1.1715x; 1.0397x over previous
import functools

import jax
import jax.numpy as jnp
from jax import lax
from jax.experimental import pallas as pl
from jax.experimental.pallas import tpu as pltpu
from jax.experimental.pallas import tpu_sc as plsc

H = 128
EPS = 1e-5
NC, NS, L = 2, 16, 16
NW = NC * NS
CH = 128
NG = 50000
QR = 12504
Q3 = NG - 3 * QR
BC = 96
CAPW = 5216
SROWS = 12576


def _bdot(a, b):
    return jnp.dot(a, b, preferred_element_type=jnp.float32)


def _mm_body(x_ref, w_ref, o_ref):
    o_ref[...] = _bdot(x_ref[...], w_ref[...])


def _matmul(x, w, bn):
    n, k = x.shape
    m = w.shape[1]
    return pl.pallas_call(
        _mm_body,
        grid=(n // bn,),
        in_specs=[pl.BlockSpec((bn, k), lambda i: (i, 0)),
                  pl.BlockSpec((k, m), lambda i: (0, 0))],
        out_specs=pl.BlockSpec((bn, m), lambda i: (i, 0)),
        out_shape=jax.ShapeDtypeStruct((n, m), jnp.float32),
    )(x, w)


def _ln_tc(h, g, b):
    mu = jnp.mean(h, axis=-1, keepdims=True)
    var = jnp.mean((h - mu) * (h - mu), axis=-1, keepdims=True)
    return (h - mu) * lax.rsqrt(var + EPS) * g + b


def _silu(x):
    return x * jax.nn.sigmoid(x)


def _edge_body(em_ref, g_ref, w1_ref, b1_ref, lg_ref, lb_ref, w2_ref, b2_ref,
               o_ref):
    em = em_ref[...]
    pre = _bdot(em, w1_ref[...]) + g_ref[...] + b1_ref[...]
    h = _ln_tc(_silu(pre), lg_ref[...], lb_ref[...])
    o_ref[...] = em + _bdot(h, w2_ref[...]) + b2_ref[...]


def _edge_mlp(em, g, w1, b1, lg, lb, w2, b2, be):
    n = em.shape[0]
    vec = lambda v: v.reshape(1, H)
    row_spec = pl.BlockSpec((1, H), lambda i: (0, 0))
    return pl.pallas_call(
        _edge_body,
        grid=(n // be,),
        in_specs=[pl.BlockSpec((be, H), lambda i: (i, 0)),
                  pl.BlockSpec((be, H), lambda i: (i, 0)),
                  pl.BlockSpec((H, H), lambda i: (0, 0)),
                  row_spec, row_spec, row_spec,
                  pl.BlockSpec((H, H), lambda i: (0, 0)),
                  row_spec],
        out_specs=pl.BlockSpec((be, H), lambda i: (i, 0)),
        out_shape=jax.ShapeDtypeStruct((n, H), jnp.float32),
    )(em, g, w1, vec(b1), vec(lg), vec(lb), w2, vec(b2))


def _node_body(vg_ref, agg_ref, w1a_ref, w1b_ref, b1_ref, lg_ref, lb_ref,
               w2_ref, b2_ref, ow1_ref, ob1_ref, ow2_ref, ob2_ref, o_ref):
    vg = vg_ref[...]
    pre = (_bdot(vg, w1a_ref[...]) + _bdot(agg_ref[...], w1b_ref[...])
           + b1_ref[...])
    h = _ln_tc(_silu(pre), lg_ref[...], lb_ref[...])
    vg2 = vg + _bdot(h, w2_ref[...]) + b2_ref[...]
    h2 = _silu(_bdot(vg2, ow1_ref[...]) + ob1_ref[...])
    o_ref[...] = _bdot(h2, ow2_ref[...]) + ob2_ref[...]


def _node_mlp(vg, agg, w1a, w1b, b1, lg, lb, w2, b2, ow1, ob1, ow2, ob2, bn):
    n = vg.shape[0]
    out = ow2.shape[1]
    vec = lambda v: v.reshape(1, -1)
    row_spec = pl.BlockSpec((1, H), lambda i: (0, 0))
    orow_spec = pl.BlockSpec((1, out), lambda i: (0, 0))
    return pl.pallas_call(
        _node_body,
        grid=(n // bn,),
        in_specs=[pl.BlockSpec((bn, H), lambda i: (i, 0)),
                  pl.BlockSpec((bn, H), lambda i: (i, 0)),
                  pl.BlockSpec((H, H), lambda i: (0, 0)),
                  pl.BlockSpec((H, H), lambda i: (0, 0)),
                  row_spec, row_spec, row_spec,
                  pl.BlockSpec((H, H), lambda i: (0, 0)),
                  row_spec,
                  pl.BlockSpec((H, out), lambda i: (0, 0)),
                  orow_spec,
                  pl.BlockSpec((H, out), lambda i: (0, 0)),
                  orow_spec],
        out_specs=pl.BlockSpec((bn, out), lambda i: (i, 0)),
        out_shape=jax.ShapeDtypeStruct((n, out), jnp.float32),
    )(vg, agg, w1a, w1b, vec(b1), vec(lg), vec(lb), w2, vec(b2),
      ow1, vec(ob1), ow2, vec(ob2))


def _sc_gather_sum(tS, tR, send_p, recv_p):
    ep = send_p.shape[0]
    nch = ep // CH
    per_w = nch // NW
    npair = per_w // 2
    mesh = plsc.VectorSubcoreMesh(core_axis_name="c", subcore_axis_name="s")

    @functools.partial(
        pl.kernel,
        out_type=jax.ShapeDtypeStruct((ep, H), jnp.float32),
        mesh=mesh,
        scratch_types=[pltpu.VMEM((per_w * CH,), jnp.int32),
                       pltpu.VMEM((per_w * CH,), jnp.int32),
                       pltpu.VMEM((CH, H), jnp.float32),
                       pltpu.VMEM((CH, H), jnp.float32),
                       pltpu.VMEM((CH, H), jnp.float32),
                       pltpu.VMEM((CH, H), jnp.float32),
                       pltpu.SemaphoreType.DMA,
                       pltpu.SemaphoreType.DMA,
                       pltpu.SemaphoreType.DMA,
                       pltpu.SemaphoreType.DMA],
    )
    def k(ts_hbm, tr_hbm, s_hbm, r_hbm, g_hbm, idxa, idxb,
          a0, b0, a1, b1, sa0, sb0, sa1, sb1):
        wid = lax.axis_index("s") * NC + lax.axis_index("c")
        wbase = wid * per_w * CH
        pltpu.sync_copy(s_hbm.at[pl.ds(wbase, per_w * CH)], idxa)
        pltpu.sync_copy(r_hbm.at[pl.ds(wbase, per_w * CH)], idxb)

        def start(i, abuf, bbuf, asem, bsem):
            off = pl.ds(i * CH, CH)
            pltpu.async_copy(ts_hbm.at[idxa.at[off]], abuf, asem)
            pltpu.async_copy(tr_hbm.at[idxb.at[off]], bbuf, bsem)

        def finish(i, abuf, bbuf, asem, bsem):
            pltpu.make_async_copy(ts_hbm.at[idxa.at[pl.ds(0, CH)]],
                                  abuf, asem).wait()
            pltpu.make_async_copy(tr_hbm.at[idxb.at[pl.ds(0, CH)]],
                                  bbuf, bsem).wait()

            @pl.loop(0, CH)
            def _(rr):
                @pl.loop(0, H, step=L)
                def _(jj):
                    slc = (pl.ds(rr, 1), pl.ds(jj, L))
                    abuf.at[slc][...] = abuf.at[slc][...] + bbuf.at[slc][...]

            pltpu.sync_copy(abuf, g_hbm.at[pl.ds(wbase + i * CH, CH)])

        start(0, a0, b0, sa0, sb0)

        @pl.loop(0, npair)
        def _(j):
            c0 = 2 * j
            start(c0 + 1, a1, b1, sa1, sb1)
            finish(c0, a0, b0, sa0, sb0)

            @pl.when(j + 1 < npair)
            def _():
                start(c0 + 2, a0, b0, sa0, sb0)

            finish(c0 + 1, a1, b1, sa1, sb1)

    return k(tS, tR, send_p, recv_p)


def _sc_bucket(recv_b):
    ep = recv_b.shape[0]
    per_w = ep // NW
    mesh = plsc.VectorSubcoreMesh(core_axis_name="c", subcore_axis_name="s")

    @functools.partial(
        pl.kernel,
        out_type=(jax.ShapeDtypeStruct((NW * 4 * CAPW,), jnp.int32),
                  jax.ShapeDtypeStruct((NW * 4 * CAPW,), jnp.int32),
                  jax.ShapeDtypeStruct((NW * L,), jnp.int32)),
        mesh=mesh,
        scratch_types=[pltpu.VMEM((per_w,), jnp.int32),
                       pltpu.VMEM((4 * CAPW,), jnp.int32),
                       pltpu.VMEM((4 * CAPW,), jnp.int32),
                       pltpu.VMEM((L,), jnp.int32),
                       pltpu.SMEM((8,), jnp.int32)],
        compiler_params=pltpu.CompilerParams(needs_layout_passes=False),
    )
    def k(r_hbm, ids_hbm, lxs_hbm, cnts_hbm, rbuf, bid, blx, cbuf, offs):
        wid = lax.axis_index("s") * NC + lax.axis_index("c")
        wbase = wid * per_w
        pltpu.sync_copy(r_hbm.at[pl.ds(wbase, per_w)], rbuf)
        lane = lax.iota(jnp.int32, L)
        for q in range(4):
            offs[q] = 0

        @pl.loop(0, per_w, step=L)
        def _(i):
            rv = rbuf.at[pl.ds(i, L)][...]
            idv = wbase + i + lane
            for q in range(4):
                lv = rv - q * QR
                cq = QR if q < 3 else Q3
                m = (lv >= 0) & (lv < cq)
                off = offs[q]
                mi = m.astype(jnp.int32)
                pos = q * CAPW + off + lax.cumsum(mi) - 1
                plsc.store_scatter(bid, [pos], idv, mask=m)
                plsc.store_scatter(blx, [pos], lv, mask=m)
                offs[q] = off + jnp.sum(mi)

        cvec = jnp.zeros((L,), jnp.int32)
        for q in range(4):
            off = offs[q]
            for t in range(BC // L):
                bid.at[pl.ds(q * CAPW + off + t * L, L)][...] = (
                    jnp.full((L,), 0, jnp.int32) + wbase)
                blx.at[pl.ds(q * CAPW + off + t * L, L)][...] = (
                    jnp.full((L,), QR, jnp.int32))
            nchq = (jnp.full((L,), BC - 1, jnp.int32) + off) // BC
            cvec = cvec + jnp.where(lane == q, nchq, 0)
            pltpu.sync_copy(bid.at[pl.ds(q * CAPW, CAPW)],
                            ids_hbm.at[pl.ds((wid * 4 + q) * CAPW, CAPW)])
            pltpu.sync_copy(blx.at[pl.ds(q * CAPW, CAPW)],
                            lxs_hbm.at[pl.ds((wid * 4 + q) * CAPW, CAPW)])
        cbuf[...] = cvec
        pltpu.sync_copy(cbuf, cnts_hbm.at[pl.ds(wid * L, L)])

    return k(recv_b)


def _sc_scatter_add(e, ids, lxs, cnts):
    mesh = plsc.VectorSubcoreMesh(core_axis_name="c", subcore_axis_name="s")

    @functools.partial(
        pl.kernel,
        out_type=jax.ShapeDtypeStruct((NG, H), jnp.float32),
        mesh=mesh,
        scratch_types=[pltpu.VMEM((BC,), jnp.int32),
                       pltpu.VMEM((BC,), jnp.int32),
                       pltpu.VMEM((BC,), jnp.int32),
                       pltpu.VMEM((BC,), jnp.int32),
                       pltpu.VMEM((L,), jnp.int32),
                       pltpu.VMEM((L,), jnp.int32),
                       pltpu.VMEM((BC, H), jnp.float32),
                       pltpu.VMEM((BC, H), jnp.float32),
                       pltpu.VMEM_SHARED((SROWS, H), jnp.float32),
                       pltpu.SemaphoreType.DMA,
                       pltpu.SemaphoreType.DMA],
        compiler_params=pltpu.CompilerParams(needs_layout_passes=False),
    )
    def k(e_hbm, ids_hbm, lxs_hbm, cnts_hbm, z_hbm, agg_hbm,
          id0, id1, lx0, lx1, cb0, cb1, e0, e1, acc, se0, se1):
        cid = lax.axis_index("c")
        sid = lax.axis_index("s")
        lane = lax.iota(jnp.int32, L)

        @pl.loop(0, 2)
        def _(p):
            q = 2 * p + cid
            lo = q * QR
            cnt = jnp.where(q < 3, QR, Q3)

            @pl.loop(sid, SROWS // BC, step=NS)
            def _(b):
                pltpu.sync_copy(z_hbm, acc.at[pl.ds(b * BC, BC)])

            plsc.subcore_barrier()

            w0 = 2 * sid
            w1 = w0 + 1
            pltpu.sync_copy(cnts_hbm.at[pl.ds(w0 * L, L)], cb0)
            pltpu.sync_copy(cnts_hbm.at[pl.ds(w1 * L, L)], cb1)
            n0 = jnp.max(jnp.where(lane == q, cb0[...], 0))
            n1 = jnp.max(jnp.where(lane == q, cb1[...], 0))
            total = n0 + n1

            def seg_off(i):
                in0 = i < n0
                w = jnp.where(in0, w0, w1)
                c = jnp.where(in0, i, i - n0)
                return (w * 4 + q) * CAPW + c * BC

            def start(i, ebuf, idb, lxb, se):
                off = seg_off(i)
                pltpu.sync_copy(ids_hbm.at[pl.ds(off, BC)], idb)
                pltpu.sync_copy(lxs_hbm.at[pl.ds(off, BC)], lxb)
                pltpu.async_copy(e_hbm.at[idb], ebuf, se)

            def finish(i, ebuf, idb, lxb, se):
                pltpu.make_async_copy(e_hbm.at[pl.ds(0, BC)], ebuf,
                                      se).wait()
                pltpu.sync_copy(ebuf, acc.at[lxb], add=True)

            npr = lax.shift_right_logical(total, 1)
            odd = total - 2 * npr

            @pl.when(total > 0)
            def _():
                start(0, e0, id0, lx0, se0)

            @pl.loop(0, npr)
            def _(j):
                c0 = 2 * j
                start(c0 + 1, e1, id1, lx1, se1)
                finish(c0, e0, id0, lx0, se0)

                @pl.when(c0 + 2 < total)
                def _():
                    start(c0 + 2, e0, id0, lx0, se0)

                finish(c0 + 1, e1, id1, lx1, se1)

            @pl.when(odd == 1)
            def _():
                finish(total - 1, e0, id0, lx0, se0)

            plsc.subcore_barrier()

            @pl.loop(sid, cnt // 8, step=NS)
            def _(b):
                pltpu.sync_copy(acc.at[pl.ds(b * 8, 8)],
                                agg_hbm.at[pl.ds(lo + b * 8, 8)])

            plsc.subcore_barrier()

    return k(e, ids, lxs, cnts, jnp.zeros((BC, H), jnp.float32))


def kernel(vM, vG, eM2G, senders, receivers, e_W1, e_b1, e_g, e_beta, e_W2,
           e_b2, n_W1, n_b1, n_g, n_beta, n_W2, n_b2, o_W1, o_b1, o_W2, o_b2):
    vm0 = vM[0]
    vg0 = vG[0]
    E = senders.shape[0]
    ep = ((E + NW * CH - 1) // (NW * CH)) * (NW * CH)
    pad = ep - E

    w1e, w1m, w1g = e_W1[:H], e_W1[H:2 * H], e_W1[2 * H:]
    nw1a, nw1b = n_W1[:H], n_W1[H:]

    send_p = jnp.concatenate([senders, jnp.zeros((pad,), jnp.int32)])
    recv_g = jnp.concatenate([receivers, jnp.zeros((pad,), jnp.int32)])
    recv_s = jnp.concatenate(
        [receivers, jnp.full((pad,), jnp.int32(1 << 30))])
    em_p = jnp.concatenate([eM2G, jnp.zeros((pad, H), jnp.float32)])

    ids, lxs, cnts = _sc_bucket(recv_s)
    tS = _matmul(vm0, w1m, 2000)
    tR = _matmul(vg0, w1g, 2000)
    G = _sc_gather_sum(tS, tR, send_p, recv_g)
    e = _edge_mlp(em_p, G, w1e, e_b1, e_g, e_beta, e_W2, e_b2, 4096)
    agg = _sc_scatter_add(e, ids, lxs, cnts)
    out = _node_mlp(vg0, agg, nw1a, nw1b, n_b1, n_g, n_beta, n_W2, n_b2,
                    o_W1, o_b1, o_W2, o_b2, 2000)
    return out[None]
